# Initial kernel scaffold; baseline (speedup 1.0000x reference)
import jax
import jax.numpy as jnp
from jax.experimental import pallas as pl


def kernel(x, edge_index, W1, b1, W2, b2, W3, b3, W4, b4, Wh, bh):
    raise NotImplementedError("write your pallas kernel here")


# speedup vs baseline: 4.4257x; 4.4257x over previous
import functools

import jax
import jax.numpy as jnp
from jax import lax
from jax.experimental import pallas as pl
from jax.experimental.pallas import tpu as pltpu
from jax.experimental.pallas import tpu_sc as plsc

N = 10000
D = 128
E = 320000
NPAD = 10240
NW = 32
CH = 128
CPT = 79
EPT = CPT * CH
EPAD = NW * EPT
ROWS_PER_SUB = NPAD // 16


def _aggregate(x_pad, src3, dst3):
    mesh = plsc.VectorSubcoreMesh(core_axis_name="c", subcore_axis_name="s")

    @functools.partial(
        pl.kernel,
        out_type=jax.ShapeDtypeStruct((2, NPAD, D), jnp.float32),
        mesh=mesh,
        scratch_types=[
            pltpu.VMEM((CPT, CH), jnp.int32),
            pltpu.VMEM((CPT, CH), jnp.int32),
            pltpu.VMEM((CH, D), jnp.float32),
            pltpu.VMEM_SHARED((NPAD, D), jnp.float32),
            pltpu.SemaphoreType.DMA,
        ],
    )
    def agg(x_hbm, src_hbm, dst_hbm, out_hbm, src_v, dst_v, rows_v, acc, sem):
        c = lax.axis_index("c")
        s = lax.axis_index("s")
        wid = s * 2 + c
        row0 = s * ROWS_PER_SUB

        pltpu.sync_copy(x_hbm.at[pl.ds(row0, ROWS_PER_SUB)],
                        acc.at[pl.ds(row0, ROWS_PER_SUB)])
        pltpu.sync_copy(src_hbm.at[wid], src_v)
        pltpu.sync_copy(dst_hbm.at[wid], dst_v)
        plsc.subcore_barrier()

        def body(j, carry):
            pltpu.async_copy(x_hbm.at[src_v.at[j]], rows_v, sem).wait()
            pltpu.sync_copy(rows_v, acc.at[dst_v.at[j]], add=True)
            return carry

        lax.fori_loop(0, CPT, body, 0)
        plsc.subcore_barrier()
        pltpu.sync_copy(acc.at[pl.ds(row0, ROWS_PER_SUB)],
                        out_hbm.at[c, pl.ds(row0, ROWS_PER_SUB)])

    return agg(x_pad, src3, dst3)


_BR = 256


def _mlp_body(p_ref, x_ref, wa, ba, wb, bb, o_ref):
    t = p_ref[0] + p_ref[1] - x_ref[...]
    h = jnp.maximum(
        jnp.dot(t, wa[...], preferred_element_type=jnp.float32) + ba[...], 0.0)
    o_ref[...] = jnp.dot(h, wb[...], preferred_element_type=jnp.float32) + bb[...]


def _mlp_head_body(p_ref, x_ref, wa, ba, wb, bb, wh, bh, o_ref):
    t = p_ref[0] + p_ref[1] - x_ref[...]
    h = jnp.maximum(
        jnp.dot(t, wa[...], preferred_element_type=jnp.float32) + ba[...], 0.0)
    g = jnp.dot(h, wb[...], preferred_element_type=jnp.float32) + bb[...]
    o_ref[...] = jnp.dot(g, wh[...], preferred_element_type=jnp.float32) + bh[...]


def _w_spec():
    return pl.BlockSpec((D, D), lambda i: (0, 0))


def _b_spec():
    return pl.BlockSpec((1, D), lambda i: (0, 0))


def _mlp(p, xin, Wa, ba, Wb, bb):
    return pl.pallas_call(
        _mlp_body,
        grid=(NPAD // _BR,),
        in_specs=[
            pl.BlockSpec((2, _BR, D), lambda i: (0, i, 0)),
            pl.BlockSpec((_BR, D), lambda i: (i, 0)),
            _w_spec(), _b_spec(), _w_spec(), _b_spec(),
        ],
        out_specs=pl.BlockSpec((_BR, D), lambda i: (i, 0)),
        out_shape=jax.ShapeDtypeStruct((NPAD, D), jnp.float32),
    )(p, xin, Wa, ba.reshape(1, D), Wb, bb.reshape(1, D))


def _mlp_head(p, xin, Wa, ba, Wb, bb, Wh, bh):
    return pl.pallas_call(
        _mlp_head_body,
        grid=(NPAD // _BR,),
        in_specs=[
            pl.BlockSpec((2, _BR, D), lambda i: (0, i, 0)),
            pl.BlockSpec((_BR, D), lambda i: (i, 0)),
            _w_spec(), _b_spec(), _w_spec(), _b_spec(), _w_spec(), _b_spec(),
        ],
        out_specs=pl.BlockSpec((_BR, D), lambda i: (i, 0)),
        out_shape=jax.ShapeDtypeStruct((NPAD, D), jnp.float32),
    )(p, xin, Wa, ba.reshape(1, D), Wb, bb.reshape(1, D), Wh, bh.reshape(1, D))


def kernel(x, edge_index, W1, b1, W2, b2, W3, b3, W4, b4, Wh, bh):
    x_pad = jnp.pad(x, ((0, NPAD - N), (0, 0)))
    fill = jnp.full((EPAD - E,), N, dtype=jnp.int32)
    src3 = jnp.concatenate([edge_index[0], fill]).reshape(NW, CPT, CH)
    dst3 = jnp.concatenate([edge_index[1], fill]).reshape(NW, CPT, CH)

    p1 = _aggregate(x_pad, src3, dst3)
    h1 = _mlp(p1, x_pad, W1, b1, W2, b2)
    p2 = _aggregate(h1, src3, dst3)
    y = _mlp_head(p2, h1, W3, b3, W4, b4, Wh, bh)
    return y[:N]
